# baseline (speedup 1.0000x reference)
import functools
import math

import jax
import jax.numpy as jnp
import numpy as np
from jax import lax
from jax.experimental import pallas as pl
from jax.experimental.pallas import tpu as pltpu
from jax.experimental.pallas import tpu_sc as plsc

MASK_TOKEN = 103
VOCAB = 30522
MULT = (2 ** 16 % VOCAB) ** 2 % VOCAB
W16 = 2 ** 16 % VOCAB
RECIP = 1.0 / VOCAB

T_MASKED = int(math.ceil(float(np.float32(0.15)) * 2 ** 23))
T_MASK = int(math.ceil(float(np.float32(0.15 * 0.8)) * 2 ** 23))
T_RAND = int(math.ceil(float(np.float32(0.15 * 0.9)) * 2 ** 23))

K_RAND = (1832780943, 270669613)
K_HI = (3187376881, 129218101)
K_LO = (2350016172, 1168365246)

_ROT_A = (13, 15, 26, 6)
_ROT_B = (17, 29, 16, 24)


def _rotl(x, d):
    return jax.lax.shift_left(x, jnp.uint32(d)) | jax.lax.shift_right_logical(
        x, jnp.uint32(32 - d))


def _threefry_bits(k1, k2, idx):
    ks0 = jnp.uint32(k1)
    ks1 = jnp.uint32(k2)
    ks2 = jnp.uint32(k1 ^ k2 ^ 0x1BD11BDA)
    ks = (ks0, ks1, ks2)
    x0 = jnp.full_like(idx, ks0)
    x1 = idx + ks1
    rots = (_ROT_A, _ROT_B, _ROT_A, _ROT_B, _ROT_A)
    for i in range(5):
        for r in rots[i]:
            x0 = x0 + x1
            x1 = _rotl(x1, r)
            x1 = x0 ^ x1
        x0 = x0 + ks[(i + 1) % 3]
        x1 = x1 + ks[(i + 2) % 3] + jnp.uint32(i + 1)
    return x0 ^ x1


def _mod_span(t):
    q = (t.astype(jnp.float32) * jnp.float32(RECIP)).astype(jnp.int32)
    r = t - q * jnp.int32(VOCAB)
    r = jnp.where(r < 0, r + jnp.int32(VOCAB), r)
    r = jnp.where(r >= jnp.int32(VOCAB), r - jnp.int32(VOCAB), r)
    return r


def _mod_span_u32(bits):
    a = jax.lax.shift_right_logical(bits, jnp.uint32(16)).astype(jnp.int32)
    b = (bits & jnp.uint32(0xFFFF)).astype(jnp.int32)
    return _mod_span(a * jnp.int32(W16) + b)


N_TOTAL = 4 * 8192
N_WORKERS = 32
CHUNK = N_TOTAL // N_WORKERS
LANES = 16
VECS = CHUNK // LANES

_mesh = plsc.VectorSubcoreMesh(core_axis_name="c", subcore_axis_name="s")


@functools.partial(
    pl.kernel,
    mesh=_mesh,
    out_type=(jax.ShapeDtypeStruct((N_TOTAL,), jnp.int32),
              jax.ShapeDtypeStruct((N_TOTAL,), jnp.int32)),
    scratch_types=[pltpu.VMEM((CHUNK,), jnp.int32),
                   pltpu.VMEM((CHUNK,), jnp.int32),
                   pltpu.VMEM((CHUNK,), jnp.int32)],
)
def _sc_mask_kernel(x_hbm, out_hbm, lab_hbm, xv, ov, lv):
    wid = lax.axis_index("s") * 2 + lax.axis_index("c")
    base = wid * CHUNK
    pltpu.sync_copy(x_hbm.at[pl.ds(base, CHUNK)], xv)
    lane = lax.iota(jnp.int32, LANES)

    def body(i, carry):
        off = i * LANES
        sl = pl.ds(off, LANES)
        xs = xv[sl]
        idx = (lane + (base + off)).astype(jnp.uint32)
        m = lax.shift_right_logical(_threefry_bits(*K_RAND, idx), jnp.uint32(9))
        masked = m < jnp.uint32(T_MASKED)
        mask_mask = m < jnp.uint32(T_MASK)
        random_mask = (m >= jnp.uint32(T_MASK)) & (m < jnp.uint32(T_RAND))
        out = jnp.where(mask_mask, jnp.int32(MASK_TOKEN), xs)
        lv[sl] = jnp.where(masked, xs, jnp.int32(-100))
        hi = _threefry_bits(*K_HI, idx)
        lo = _threefry_bits(*K_LO, idx)
        toks = _mod_span(
            _mod_span_u32(hi) * jnp.int32(MULT) + _mod_span_u32(lo))
        ov[sl] = jnp.where(random_mask, toks, out)
        return carry

    lax.fori_loop(0, VECS, body, 0, unroll=4)
    pltpu.sync_copy(ov, out_hbm.at[pl.ds(base, CHUNK)])
    pltpu.sync_copy(lv, lab_hbm.at[pl.ds(base, CHUNK)])


def kernel(x):
    shape = x.shape
    out, lab = _sc_mask_kernel(x.reshape(-1))
    return out.reshape(shape), lab.reshape(shape)


# speedup vs baseline: 7.4398x; 7.4398x over previous
import math

import jax
import jax.numpy as jnp
import numpy as np
from jax.experimental import pallas as pl

MASK_TOKEN = 103
VOCAB = 30522
MULT = (2 ** 16 % VOCAB) ** 2 % VOCAB
W16 = 2 ** 16 % VOCAB
RECIP = 1.0 / VOCAB

T_MASKED = int(math.ceil(float(np.float32(0.15)) * 2 ** 23))
T_MASK = int(math.ceil(float(np.float32(0.15 * 0.8)) * 2 ** 23))
T_RAND = int(math.ceil(float(np.float32(0.15 * 0.9)) * 2 ** 23))

K_RAND = (1832780943, 270669613)
K_HI = (3187376881, 129218101)
K_LO = (2350016172, 1168365246)

_ROT_A = (13, 15, 26, 6)
_ROT_B = (17, 29, 16, 24)

ROWS, COLS = 4, 8192


def _rotl(x, d):
    return jax.lax.shift_left(x, jnp.uint32(d)) | jax.lax.shift_right_logical(
        x, jnp.uint32(32 - d))


def _threefry_bits(k1, k2, idx):
    ks0 = jnp.uint32(k1)
    ks1 = jnp.uint32(k2)
    ks2 = jnp.uint32(k1 ^ k2 ^ 0x1BD11BDA)
    ks = (ks0, ks1, ks2)
    x0 = jnp.full_like(idx, ks0)
    x1 = idx + ks1
    rots = (_ROT_A, _ROT_B, _ROT_A, _ROT_B, _ROT_A)
    for i in range(5):
        for r in rots[i]:
            x0 = x0 + x1
            x1 = _rotl(x1, r)
            x1 = x0 ^ x1
        x0 = x0 + ks[(i + 1) % 3]
        x1 = x1 + ks[(i + 2) % 3] + jnp.uint32(i + 1)
    return x0 ^ x1


def _mod_span(t):
    q = (t.astype(jnp.float32) * jnp.float32(RECIP)).astype(jnp.int32)
    r = t - q * jnp.int32(VOCAB)
    r = jnp.where(r < 0, r + jnp.int32(VOCAB), r)
    r = jnp.where(r >= jnp.int32(VOCAB), r - jnp.int32(VOCAB), r)
    return r


def _mod_span_u32(bits):
    a = jax.lax.shift_right_logical(bits, jnp.uint32(16)).astype(jnp.int32)
    b = (bits & jnp.uint32(0xFFFF)).astype(jnp.int32)
    return _mod_span(a * jnp.int32(W16) + b)


def _mask_kernel(x_ref, out_ref, lab_ref):
    rows, cols = x_ref.shape
    half = cols // 2
    row = jax.lax.broadcasted_iota(jnp.uint32, (2 * rows, half), 0)
    col = jax.lax.broadcasted_iota(jnp.uint32, (2 * rows, half), 1)
    idx = ((row & jnp.uint32(3)) * jnp.uint32(cols)
           + jax.lax.shift_right_logical(row, jnp.uint32(2)) * jnp.uint32(half)
           + col)

    m = jax.lax.shift_right_logical(_threefry_bits(*K_RAND, idx), jnp.uint32(9))
    masked = m < jnp.uint32(T_MASKED)
    mask_mask = m < jnp.uint32(T_MASK)
    random_mask = (m >= jnp.uint32(T_MASK)) & (m < jnp.uint32(T_RAND))
    hi = _threefry_bits(*K_HI, idx)
    lo = _threefry_bits(*K_LO, idx)
    toks = _mod_span(_mod_span_u32(hi) * jnp.int32(MULT) + _mod_span_u32(lo))

    rep = jnp.where(mask_mask, jnp.int32(MASK_TOKEN),
                    jnp.where(random_mask, toks, jnp.int32(-1)))

    for h in range(2):
        reph = rep[h * rows:(h + 1) * rows, :]
        mh = masked[h * rows:(h + 1) * rows, :]
        xs = x_ref[:, h * half:(h + 1) * half]
        lab_ref[:, h * half:(h + 1) * half] = jnp.where(
            mh, xs, jnp.int32(-100))
        out_ref[:, h * half:(h + 1) * half] = jnp.where(
            reph >= jnp.int32(0), reph, xs)


def kernel(x):
    out_shape = jax.ShapeDtypeStruct(x.shape, x.dtype)
    return pl.pallas_call(
        _mask_kernel,
        out_shape=(out_shape, out_shape),
    )(x)
